# Initial kernel scaffold; baseline (speedup 1.0000x reference)
import jax
import jax.numpy as jnp
from jax import lax
from jax.experimental import pallas as pl
from jax.experimental.pallas import tpu as pltpu
from jax.experimental.pallas import tpu_sc as plsc

_N = 10000
_E = 320000
_G = 16

_NTILES = 16
_NCORES = 2
_EPW = _E // (_NCORES * _NTILES)
_CH = 80
_NCHUNK = _EPW // _CH
_RPT = _N // _NTILES
_RCH = 125
_NRC = _RPT // _RCH


def _fc1_body(x_ref, w_ref, b_ref, o_ref):
    o_ref[...] = jnp.maximum(
        jnp.dot(x_ref[...], w_ref[...], preferred_element_type=jnp.float32)
        + b_ref[...], 0.0)


def _fc1(xp, W1p, b1r):
    nb = 2000
    return pl.pallas_call(
        _fc1_body,
        grid=(_N // nb,),
        in_specs=[
            pl.BlockSpec((nb, 128), lambda i: (i, 0)),
            pl.BlockSpec((128, 128), lambda i: (0, 0)),
            pl.BlockSpec((1, 128), lambda i: (0, 0)),
        ],
        out_specs=pl.BlockSpec((nb, 128), lambda i: (i, 0)),
        out_shape=jax.ShapeDtypeStruct((_N, 128), jnp.float32),
    )(xp, W1p, b1r)


def _segsum_body(h_hbm, src_hbm, dst_hbm, z_hbm, out_hbm,
                 acc, src_v, dst_v, rows_v, obuf, sem):
    c = lax.axis_index("c")
    s = lax.axis_index("s")
    w = c * _NTILES + s
    rbase = s * _RPT

    pltpu.sync_copy(z_hbm, obuf)

    def zk(k, carry):
        pltpu.sync_copy(obuf, acc.at[pl.ds(rbase + k * _RCH, _RCH)])
        return carry
    lax.fori_loop(0, _NRC, zk, 0)
    plsc.subcore_barrier()

    ebase = w * _EPW

    def step(j, carry):
        b = ebase + j * _CH
        pltpu.sync_copy(src_hbm.at[pl.ds(b, _CH)], src_v)
        pltpu.sync_copy(dst_hbm.at[pl.ds(b, _CH)], dst_v)
        pltpu.async_copy(h_hbm.at[src_v], rows_v, sem).wait()
        pltpu.sync_copy(rows_v, acc.at[dst_v], add=True)
        return carry
    lax.fori_loop(0, _NCHUNK, step, 0)
    plsc.subcore_barrier()

    def ok(k, carry):
        r = rbase + k * _RCH
        pltpu.sync_copy(acc.at[pl.ds(r, _RCH)], obuf)
        pltpu.sync_copy(obuf, out_hbm.at[c, pl.ds(r, _RCH)])
        return carry
    lax.fori_loop(0, _NRC, ok, 0)


def _segsum(h, src, dst, zrows):
    mesh = plsc.VectorSubcoreMesh(core_axis_name="c", subcore_axis_name="s")
    f = pl.kernel(
        _segsum_body,
        mesh=mesh,
        out_type=jax.ShapeDtypeStruct((_NCORES, _N, 128), jnp.float32),
        scratch_types=[
            pltpu.VMEM_SHARED((_N, 128), jnp.float32),
            pltpu.VMEM((_CH,), jnp.int32),
            pltpu.VMEM((_CH,), jnp.int32),
            pltpu.VMEM((_CH, 128), jnp.float32),
            pltpu.VMEM((_RCH, 128), jnp.float32),
            pltpu.SemaphoreType.DMA,
        ],
    )
    return f(h, src, dst, zrows)


def _tail_body(h_ref, s0_ref, s1_ref, bb_ref, wr_ref, wn_ref, bc_ref,
               w2_ref, b2_ref, wmu_ref, bmu_ref, wlv_ref, blv_ref,
               mu_ref, lv_ref, acc_ref):
    i = pl.program_id(0)
    nsteps = pl.num_programs(0)

    @pl.when(i == 0)
    def _():
        acc_ref[...] = jnp.full((_G, 128), -jnp.inf, jnp.float32)

    h = h_ref[...]
    sagg = s0_ref[...] + s1_ref[...]
    h2 = (jnp.dot(h, wr_ref[...], preferred_element_type=jnp.float32)
          + jnp.dot(sagg, wn_ref[...], preferred_element_type=jnp.float32)
          + bc_ref[...])
    h3 = jnp.maximum(
        jnp.dot(h2, w2_ref[...], preferred_element_type=jnp.float32)
        + b2_ref[...], 0.0)

    bb = bb_ref[...]
    for g in range(_G):
        red = jnp.max(jnp.where(bb == g, h3, -jnp.inf), axis=0,
                      keepdims=True)
        acc_ref[pl.ds(g, 1), :] = jnp.maximum(acc_ref[pl.ds(g, 1), :], red)

    @pl.when(i == nsteps - 1)
    def _():
        pooled = acc_ref[...]
        mu_ref[...] = (jnp.dot(pooled, wmu_ref[...],
                               preferred_element_type=jnp.float32)
                       + bmu_ref[...])
        lv_ref[...] = (jnp.dot(pooled, wlv_ref[...],
                               preferred_element_type=jnp.float32)
                       + blv_ref[...])


def _tail(h, s0, s1, bb, Wroot, Wnbr, bcr, W2, b2r, Wmu, bmur, Wlv, blvr):
    nb = 2000
    full = lambda r, c: pl.BlockSpec((r, c), lambda i: (0, 0))
    return pl.pallas_call(
        _tail_body,
        grid=(_N // nb,),
        in_specs=[
            pl.BlockSpec((nb, 128), lambda i: (i, 0)),
            pl.BlockSpec((nb, 128), lambda i: (i, 0)),
            pl.BlockSpec((nb, 128), lambda i: (i, 0)),
            pl.BlockSpec((nb, 128), lambda i: (i, 0)),
            full(128, 256),
            full(128, 256),
            full(1, 256),
            full(256, 128),
            full(1, 128),
            full(128, 128),
            full(1, 128),
            full(128, 128),
            full(1, 128),
        ],
        out_specs=[full(_G, 128), full(_G, 128)],
        out_shape=[jax.ShapeDtypeStruct((_G, 128), jnp.float32),
                   jax.ShapeDtypeStruct((_G, 128), jnp.float32)],
        scratch_shapes=[pltpu.VMEM((_G, 128), jnp.float32)],
    )(h, s0, s1, bb, Wroot, Wnbr, bcr, W2, b2r, Wmu, bmur, Wlv, blvr)


def kernel(x, edge_index, batch, W1, b1, Wroot, Wnbr, bconv, W2, b2,
           Wmu, bmu, Wlv, blv):
    src = edge_index[0]
    dst = edge_index[1]
    xp = jnp.pad(x, ((0, 0), (0, 125)))
    W1p = jnp.pad(W1, ((0, 125), (0, 0)))
    h = _fc1(xp, W1p, b1.reshape(1, 128))
    zrows = jnp.zeros((_RCH, 128), jnp.float32)
    partials = _segsum(h, src, dst, zrows)
    bb = jnp.broadcast_to(batch[:, None], (_N, 128))
    mu, lv = _tail(h, partials[0], partials[1], bb, Wroot, Wnbr,
                   bconv.reshape(1, 256), W2, b2.reshape(1, 128),
                   Wmu, bmu.reshape(1, 128), Wlv, blv.reshape(1, 128))
    return (mu, lv)


# speedup vs baseline: 6.3230x; 6.3230x over previous
import jax
import jax.numpy as jnp
from jax import lax
from jax.experimental import pallas as pl
from jax.experimental.pallas import tpu as pltpu
from jax.experimental.pallas import tpu_sc as plsc

_N = 10000
_E = 320000
_G = 16

_NTILES = 16
_NCORES = 2
_EPW = _E // (_NCORES * _NTILES)
_CH = 80
_NCHUNK = _EPW // _CH
_NPAD = 10240
_RPT = _NPAD // _NTILES
_RCH = 128
_NRC = _RPT // _RCH


def _fc1_body(x_ref, w_ref, b_ref, o_ref):
    o_ref[...] = jnp.maximum(
        jnp.dot(x_ref[...], w_ref[...], preferred_element_type=jnp.float32)
        + b_ref[...], 0.0)


def _fc1(xp, W1p, b1r):
    nb = 2000
    return pl.pallas_call(
        _fc1_body,
        grid=(_N // nb,),
        in_specs=[
            pl.BlockSpec((nb, 128), lambda i: (i, 0)),
            pl.BlockSpec((128, 128), lambda i: (0, 0)),
            pl.BlockSpec((1, 128), lambda i: (0, 0)),
        ],
        out_specs=pl.BlockSpec((nb, 128), lambda i: (i, 0)),
        out_shape=jax.ShapeDtypeStruct((_N, 128), jnp.float32),
    )(xp, W1p, b1r)


def _segsum_body(h_hbm, src_hbm, dst_hbm, z_hbm, out_hbm,
                 acc, src_v, dst_v, rows_v, obuf, sem):
    c = lax.axis_index("c")
    s = lax.axis_index("s")
    w = c * _NTILES + s
    rbase = s * _RPT

    pltpu.sync_copy(z_hbm, obuf)

    def zk(k, carry):
        pltpu.sync_copy(obuf, acc.at[pl.ds(rbase + k * _RCH, _RCH)])
        return carry
    lax.fori_loop(0, _NRC, zk, 0)
    plsc.subcore_barrier()

    ebase = w * _EPW

    def step(j, carry):
        b = ebase + j * _CH
        pltpu.sync_copy(src_hbm.at[pl.ds(b, _CH)], src_v)
        pltpu.sync_copy(dst_hbm.at[pl.ds(b, _CH)], dst_v)
        pltpu.async_copy(h_hbm.at[src_v], rows_v, sem).wait()
        pltpu.sync_copy(rows_v, acc.at[dst_v], add=True)
        return carry
    lax.fori_loop(0, _NCHUNK, step, 0)
    plsc.subcore_barrier()

    def ok(k, carry):
        r = rbase + k * _RCH
        pltpu.sync_copy(acc.at[pl.ds(r, _RCH)], obuf)
        pltpu.sync_copy(obuf, out_hbm.at[c, pl.ds(r, _RCH)])
        return carry
    lax.fori_loop(0, _NRC, ok, 0)


def _segsum(h, src, dst, zrows):
    mesh = plsc.VectorSubcoreMesh(core_axis_name="c", subcore_axis_name="s")
    f = pl.kernel(
        _segsum_body,
        mesh=mesh,
        out_type=jax.ShapeDtypeStruct((_NCORES, _NPAD, 128), jnp.float32),
        scratch_types=[
            pltpu.VMEM_SHARED((_NPAD, 128), jnp.float32),
            pltpu.VMEM((_CH,), jnp.int32),
            pltpu.VMEM((_CH,), jnp.int32),
            pltpu.VMEM((_CH, 128), jnp.float32),
            pltpu.VMEM((_RCH, 128), jnp.float32),
            pltpu.SemaphoreType.DMA,
        ],
    )
    return f(h, src, dst, zrows)


def _tail_body(h_ref, s0_ref, s1_ref, bb_ref, wr_ref, wn_ref, bc_ref,
               w2_ref, b2_ref, wmu_ref, bmu_ref, wlv_ref, blv_ref,
               mu_ref, lv_ref, acc_ref):
    i = pl.program_id(0)
    nsteps = pl.num_programs(0)

    @pl.when(i == 0)
    def _():
        acc_ref[...] = jnp.full((_G, 128), -jnp.inf, jnp.float32)

    h = h_ref[...]
    sagg = s0_ref[...] + s1_ref[...]
    h2 = (jnp.dot(h, wr_ref[...], preferred_element_type=jnp.float32)
          + jnp.dot(sagg, wn_ref[...], preferred_element_type=jnp.float32)
          + bc_ref[...])
    h3 = jnp.maximum(
        jnp.dot(h2, w2_ref[...], preferred_element_type=jnp.float32)
        + b2_ref[...], 0.0)

    bb = bb_ref[...]
    for g in range(_G):
        red = jnp.max(jnp.where(bb == g, h3, -jnp.inf), axis=0,
                      keepdims=True)
        acc_ref[pl.ds(g, 1), :] = jnp.maximum(acc_ref[pl.ds(g, 1), :], red)

    @pl.when(i == nsteps - 1)
    def _():
        pooled = acc_ref[...]
        mu_ref[...] = (jnp.dot(pooled, wmu_ref[...],
                               preferred_element_type=jnp.float32)
                       + bmu_ref[...])
        lv_ref[...] = (jnp.dot(pooled, wlv_ref[...],
                               preferred_element_type=jnp.float32)
                       + blv_ref[...])


def _tail(h, s0, s1, bb, Wroot, Wnbr, bcr, W2, b2r, Wmu, bmur, Wlv, blvr):
    nb = 2000
    full = lambda r, c: pl.BlockSpec((r, c), lambda i: (0, 0))
    return pl.pallas_call(
        _tail_body,
        grid=(_N // nb,),
        in_specs=[
            pl.BlockSpec((nb, 128), lambda i: (i, 0)),
            pl.BlockSpec((nb, 128), lambda i: (i, 0)),
            pl.BlockSpec((nb, 128), lambda i: (i, 0)),
            pl.BlockSpec((nb, 128), lambda i: (i, 0)),
            full(128, 256),
            full(128, 256),
            full(1, 256),
            full(256, 128),
            full(1, 128),
            full(128, 128),
            full(1, 128),
            full(128, 128),
            full(1, 128),
        ],
        out_specs=[full(_G, 128), full(_G, 128)],
        out_shape=[jax.ShapeDtypeStruct((_G, 128), jnp.float32),
                   jax.ShapeDtypeStruct((_G, 128), jnp.float32)],
        scratch_shapes=[pltpu.VMEM((_G, 128), jnp.float32)],
    )(h, s0, s1, bb, Wroot, Wnbr, bcr, W2, b2r, Wmu, bmur, Wlv, blvr)


def kernel(x, edge_index, batch, W1, b1, Wroot, Wnbr, bconv, W2, b2,
           Wmu, bmu, Wlv, blv):
    src = edge_index[0]
    dst = edge_index[1]
    xp = jnp.pad(x, ((0, 0), (0, 125)))
    W1p = jnp.pad(W1, ((0, 125), (0, 0)))
    h = _fc1(xp, W1p, b1.reshape(1, 128))
    zrows = jnp.zeros((_RCH, 128), jnp.float32)
    partials = _segsum(h, src, dst, zrows)[:, :_N, :]
    bb = jnp.broadcast_to(batch[:, None], (_N, 128))
    mu, lv = _tail(h, partials[0], partials[1], bb, Wroot, Wnbr,
                   bconv.reshape(1, 256), W2, b2.reshape(1, 128),
                   Wmu, bmu.reshape(1, 128), Wlv, blv.reshape(1, 128))
    return (mu, lv)
